# baseline (speedup 1.0000x reference)
import functools

import jax
import jax.numpy as jnp
from jax import lax
from jax.experimental import pallas as pl
from jax.experimental.pallas import tpu as pltpu
from jax.experimental.pallas import tpu_sc as plsc

N_NODES = 100000
H = 64
MSG = 68
B = 16384

NC = 2
NS = 16
L = 16
NW = NC * NS
OWN_W = 3128
OWN_LAST = N_NODES - (NW - 1) * OWN_W
NPIECE = 25
TPAD = NPIECE * 128
B_W = B // NW
BG = B // L

_mesh = plsc.VectorSubcoreMesh(core_axis_name="c", subcore_axis_name="s")
_sc_params = pltpu.CompilerParams(use_tc_tiling_on_sc=False,
                                  needs_layout_passes=False)


def _worker_id():
    return lax.axis_index("s") * NC + lax.axis_index("c")


def _gather_body(mem_hbm, ids_hbm, out_hbm, idx_v, rows_v, sem):
    wid = _worker_id()
    pltpu.sync_copy(ids_hbm.at[pl.ds(wid * 4, 4)], idx_v)
    for j in range(4):
        pltpu.async_copy(mem_hbm.at[idx_v.at[j]], rows_v, sem).wait()
        pltpu.sync_copy(rows_v, out_hbm.at[pl.ds(wid * B_W + j * 128, 128)])


def _sc_gather(mem, ids2d):
    return pl.kernel(
        _gather_body,
        out_type=jax.ShapeDtypeStruct((B, H), jnp.float32),
        mesh=_mesh,
        scratch_types=[
            pltpu.VMEM((4, 128), jnp.int32),
            pltpu.VMEM((128, H), jnp.float32),
            pltpu.SemaphoreType.DMA,
        ],
        compiler_params=_sc_params,
    )(mem, ids2d)


def _gru_body(h_ref, m_ref, wir, wiz, win, whr, whz, whn,
              bir, biz, bin_, bhr, bhz, bhn, o_ref):
    h = h_ref[...]
    m = m_ref[...]
    dot = functools.partial(jnp.dot, preferred_element_type=jnp.float32)
    i_r = dot(m, wir[...]) + bir[...]
    i_z = dot(m, wiz[...]) + biz[...]
    i_n = dot(m, win[...]) + bin_[...]
    h_r = dot(h, whr[...]) + bhr[...]
    h_z = dot(h, whz[...]) + bhz[...]
    h_n = dot(h, whn[...]) + bhn[...]
    r = jax.nn.sigmoid(i_r + h_r)
    z = jax.nn.sigmoid(i_z + h_z)
    n = jnp.tanh(i_n + r * h_n)
    o_ref[...] = (1.0 - z) * n + z * h


def _tc_gru(h, messages, W_ih, W_hh, b_ih, b_hh):
    BT = 2048
    grid = B // BT
    wi = W_ih.T
    wh = W_hh.T
    row = lambda i: (i, 0)
    rep = lambda i: (0, 0)
    ws = [wi[:, :H], wi[:, H:2 * H], wi[:, 2 * H:],
          wh[:, :H], wh[:, H:2 * H], wh[:, 2 * H:]]
    bs = [b_ih[:H], b_ih[H:2 * H], b_ih[2 * H:],
          b_hh[:H], b_hh[H:2 * H], b_hh[2 * H:]]
    bs = [b.reshape(1, H) for b in bs]
    return pl.pallas_call(
        _gru_body,
        grid=(grid,),
        in_specs=[
            pl.BlockSpec((BT, H), row),
            pl.BlockSpec((BT, MSG), row),
            *[pl.BlockSpec((MSG, H), rep)] * 3,
            *[pl.BlockSpec((H, H), rep)] * 3,
            *[pl.BlockSpec((1, H), rep)] * 6,
        ],
        out_specs=pl.BlockSpec((BT, H), row),
        out_shape=jax.ShapeDtypeStruct((B, H), jnp.float32),
    )(h, messages, *ws, *bs)


def _scatter_body(ids_hbm, ts_hbm, lu_hbm, nh_hbm, mem_ref, out_lu,
                  idsv, tsv, table, dlist, luo, luv, rows_v,
                  sem_g, sem_s):
    wid = _worker_id()
    last = wid == NW - 1
    rbase = wid * OWN_W
    nown = jnp.where(last, OWN_LAST, OWN_W)
    lane = lax.iota(jnp.int32, 16)

    @pl.when(jnp.logical_not(last))
    def _():
        pltpu.sync_copy(lu_hbm.at[pl.ds(rbase, OWN_W)], luo.at[pl.ds(0, OWN_W)])

    @pl.when(last)
    def _():
        pltpu.sync_copy(lu_hbm.at[pl.ds(rbase, OWN_LAST)],
                        luo.at[pl.ds(0, OWN_LAST)])

    pltpu.sync_copy(ids_hbm, idsv)
    pltpu.sync_copy(ts_hbm, tsv)

    neg1 = jnp.full((16,), -1, jnp.int32)
    def init_body(t, c):
        for k in range(8):
            table[pl.ds(t * 128 + k * 16, 16)] = neg1
        return c
    lax.fori_loop(0, TPAD // 128, init_body, 0)

    UNROLL = 8
    def scan_body(it, c):
        base = it * (16 * UNROLL)
        for k in range(UNROLL):
            off = base + k * 16
            ids16 = idsv[pl.ds(off, 16)]
            loc = ids16 - rbase
            inr = (loc >= 0) & (loc < nown)
            dc = jnp.where(inr, loc, 0)
            plsc.store_scatter(table, [dc], off + lane, mask=inr)
        return c
    lax.fori_loop(0, BG // UNROLL, scan_body, 0)

    def dl_body(p, c):
        for k in range(8):
            off = p * 128 + k * 16
            tb = table[pl.ds(off, 16)]
            hit = tb >= 0
            dlist[p, pl.ds(k * 16, 16)] = jnp.where(hit, rbase + off + lane, -1)
            tsw = plsc.load_gather(tsv, [jnp.maximum(tb, 0)])
            luv[pl.ds(off, 16)] = jnp.where(hit, tsw, luo[pl.ds(off, 16)])
        return c
    lax.fori_loop(0, NPIECE, dl_body, 0)

    @pl.when(jnp.logical_not(last))
    def _():
        pltpu.sync_copy(luv.at[pl.ds(0, OWN_W)], out_lu.at[pl.ds(rbase, OWN_W)])

    @pl.when(last)
    def _():
        pltpu.sync_copy(luv.at[pl.ds(0, OWN_LAST)],
                        out_lu.at[pl.ds(rbase, OWN_LAST)])

    D = 3
    def gstart(p):
        src = plsc.Indices(table.at[pl.ds(p * 128, 128)], ignored_value=-1)
        return pltpu.async_copy(nh_hbm.at[src], rows_v.at[p % D], sem_g)

    gh = [None] * NPIECE
    sh = [None] * NPIECE
    gh[0] = gstart(0)
    gh[1] = gstart(1)
    for p in range(NPIECE):
        gh[p].wait()
        dst = plsc.Indices(dlist.at[p], ignored_value=-1)
        sh[p] = pltpu.async_copy(rows_v.at[p % D], mem_ref.at[dst], sem_s)
        q = p + 2
        if q < NPIECE:
            if q >= D:
                sh[q - D].wait()
            gh[q] = gstart(q)
    for p in range(NPIECE - D, NPIECE):
        sh[p].wait()


def _sc_scatter_inplace(node_ids, timestamps, last_update, new_h, mem_ref):
    return pl.kernel(
        _scatter_body,
        out_type=jax.ShapeDtypeStruct((N_NODES,), jnp.float32),
        mesh=_mesh,
        scratch_types=[
            pltpu.VMEM((B,), jnp.int32),
            pltpu.VMEM((B,), jnp.float32),
            pltpu.VMEM((TPAD,), jnp.int32),
            pltpu.VMEM((NPIECE, 128), jnp.int32),
            pltpu.VMEM((TPAD,), jnp.float32),
            pltpu.VMEM((TPAD,), jnp.float32),
            pltpu.VMEM((3, 128, H), jnp.float32),
            pltpu.SemaphoreType.DMA,
            pltpu.SemaphoreType.DMA,
        ],
        compiler_params=_sc_params,
    )(node_ids, timestamps, last_update, new_h, mem_ref)


def kernel(mem, node_ids, messages, timestamps, last_update,
           W_ih, W_hh, b_ih, b_hh):
    mem_ref = jax.new_ref(mem)
    ids2d = node_ids.reshape(128, 128)
    h = _sc_gather(mem, ids2d)
    new_h = _tc_gru(h, messages, W_ih, W_hh, b_ih, b_hh)
    lu_new = _sc_scatter_inplace(node_ids, timestamps, last_update, new_h,
                                 mem_ref)
    return mem_ref[...], lu_new


# speedup vs baseline: 1.0026x; 1.0026x over previous
import functools

import jax
import jax.numpy as jnp
from jax import lax
from jax.experimental import pallas as pl
from jax.experimental.pallas import tpu as pltpu
from jax.experimental.pallas import tpu_sc as plsc

N_NODES = 100000
H = 64
MSG = 68
B = 16384

NC = 2
NS = 16
L = 16
NW = NC * NS
OWN_W = 3128
OWN_LAST = N_NODES - (NW - 1) * OWN_W
NPIECE = 25
TPAD = NPIECE * 128
B_W = B // NW
BG = B // L

_mesh = plsc.VectorSubcoreMesh(core_axis_name="c", subcore_axis_name="s")
_sc_params = pltpu.CompilerParams(use_tc_tiling_on_sc=False,
                                  needs_layout_passes=False)


def _worker_id():
    return lax.axis_index("s") * NC + lax.axis_index("c")


def _gather_body(mem_hbm, ids_hbm, out_hbm, idx_v, rows_v, sem):
    wid = _worker_id()
    pltpu.sync_copy(ids_hbm.at[pl.ds(wid * 4, 4)], idx_v)
    for j in range(4):
        pltpu.async_copy(mem_hbm.at[idx_v.at[j]], rows_v, sem).wait()
        pltpu.sync_copy(rows_v, out_hbm.at[pl.ds(wid * B_W + j * 128, 128)])


def _sc_gather(mem, ids2d):
    return pl.kernel(
        _gather_body,
        out_type=jax.ShapeDtypeStruct((B, H), jnp.float32),
        mesh=_mesh,
        scratch_types=[
            pltpu.VMEM((4, 128), jnp.int32),
            pltpu.VMEM((128, H), jnp.float32),
            pltpu.SemaphoreType.DMA,
        ],
        compiler_params=_sc_params,
    )(mem, ids2d)


def _gru_body(h_ref, m_ref, wir, wiz, win, whr, whz, whn,
              bir, biz, bin_, bhr, bhz, bhn, o_ref):
    h = h_ref[...]
    m = m_ref[...]
    dot = functools.partial(jnp.dot, preferred_element_type=jnp.float32)
    i_r = dot(m, wir[...]) + bir[...]
    i_z = dot(m, wiz[...]) + biz[...]
    i_n = dot(m, win[...]) + bin_[...]
    h_r = dot(h, whr[...]) + bhr[...]
    h_z = dot(h, whz[...]) + bhz[...]
    h_n = dot(h, whn[...]) + bhn[...]
    r = jax.nn.sigmoid(i_r + h_r)
    z = jax.nn.sigmoid(i_z + h_z)
    n = jnp.tanh(i_n + r * h_n)
    o_ref[...] = (1.0 - z) * n + z * h


def _tc_gru(h, messages, W_ih, W_hh, b_ih, b_hh):
    BT = 2048
    grid = B // BT
    wi = W_ih.T
    wh = W_hh.T
    row = lambda i: (i, 0)
    rep = lambda i: (0, 0)
    ws = [wi[:, :H], wi[:, H:2 * H], wi[:, 2 * H:],
          wh[:, :H], wh[:, H:2 * H], wh[:, 2 * H:]]
    bs = [b_ih[:H], b_ih[H:2 * H], b_ih[2 * H:],
          b_hh[:H], b_hh[H:2 * H], b_hh[2 * H:]]
    bs = [b.reshape(1, H) for b in bs]
    return pl.pallas_call(
        _gru_body,
        grid=(grid,),
        in_specs=[
            pl.BlockSpec((BT, H), row),
            pl.BlockSpec((BT, MSG), row),
            *[pl.BlockSpec((MSG, H), rep)] * 3,
            *[pl.BlockSpec((H, H), rep)] * 3,
            *[pl.BlockSpec((1, H), rep)] * 6,
        ],
        out_specs=pl.BlockSpec((BT, H), row),
        out_shape=jax.ShapeDtypeStruct((B, H), jnp.float32),
    )(h, messages, *ws, *bs)


def _scatter_body(ids_hbm, ts_hbm, lu_hbm, nh_hbm, mem_ref, out_lu,
                  idsv, tsv, table, dlist, luo, luv, rows_v,
                  sem_g, sem_s):
    wid = _worker_id()
    last = wid == NW - 1
    rbase = wid * OWN_W
    nown = jnp.where(last, OWN_LAST, OWN_W)
    lane = lax.iota(jnp.int32, 16)

    @pl.when(jnp.logical_not(last))
    def _():
        pltpu.sync_copy(lu_hbm.at[pl.ds(rbase, OWN_W)], luo.at[pl.ds(0, OWN_W)])

    @pl.when(last)
    def _():
        pltpu.sync_copy(lu_hbm.at[pl.ds(rbase, OWN_LAST)],
                        luo.at[pl.ds(0, OWN_LAST)])

    pltpu.sync_copy(ids_hbm, idsv)
    pltpu.sync_copy(ts_hbm, tsv)

    neg1 = jnp.full((16,), -1, jnp.int32)
    def init_body(t, c):
        for k in range(8):
            table[pl.ds(t * 128 + k * 16, 16)] = neg1
        return c
    lax.fori_loop(0, TPAD // 128, init_body, 0)

    UNROLL = 8
    def scan_body(it, c):
        base = it * (16 * UNROLL)
        for k in range(UNROLL):
            off = base + k * 16
            ids16 = idsv[pl.ds(off, 16)]
            loc = ids16 - rbase
            inr = (loc >= 0) & (loc < nown)
            dc = jnp.where(inr, loc, 0)
            plsc.store_scatter(table, [dc], off + lane, mask=inr)
        return c
    lax.fori_loop(0, BG // UNROLL, scan_body, 0)

    def dl_body(p, c):
        for k in range(8):
            off = p * 128 + k * 16
            tb = table[pl.ds(off, 16)]
            hit = tb >= 0
            dlist[p, pl.ds(k * 16, 16)] = jnp.where(hit, rbase + off + lane, -1)
            tsw = plsc.load_gather(tsv, [jnp.maximum(tb, 0)])
            luv[pl.ds(off, 16)] = jnp.where(hit, tsw, luo[pl.ds(off, 16)])
        return c
    lax.fori_loop(0, NPIECE, dl_body, 0)

    @pl.when(jnp.logical_not(last))
    def _():
        pltpu.sync_copy(luv.at[pl.ds(0, OWN_W)], out_lu.at[pl.ds(rbase, OWN_W)])

    @pl.when(last)
    def _():
        pltpu.sync_copy(luv.at[pl.ds(0, OWN_LAST)],
                        out_lu.at[pl.ds(rbase, OWN_LAST)])

    D = 3
    def gstart(p):
        src = plsc.Indices(table.at[pl.ds(p * 128, 128)], ignored_value=-1)
        return pltpu.async_copy(nh_hbm.at[src], rows_v.at[p % D], sem_g)

    gh = [None] * NPIECE
    sh = [None] * NPIECE
    gh[0] = gstart(0)
    gh[1] = gstart(1)
    for p in range(NPIECE):
        gh[p].wait()
        dst = plsc.Indices(dlist.at[p], ignored_value=-1)
        sh[p] = pltpu.async_copy(rows_v.at[p % D], mem_ref.at[dst], sem_s)
        q = p + 2
        if q < NPIECE:
            if q >= D:
                sh[q - D].wait()
            gh[q] = gstart(q)
    for p in range(NPIECE - D, NPIECE):
        sh[p].wait()


def _sc_scatter_inplace(node_ids, timestamps, last_update, new_h, mem_ref):
    return pl.kernel(
        _scatter_body,
        out_type=jax.ShapeDtypeStruct((N_NODES,), jnp.float32),
        mesh=_mesh,
        scratch_types=[
            pltpu.VMEM((B,), jnp.int32),
            pltpu.VMEM((B,), jnp.float32),
            pltpu.VMEM((TPAD,), jnp.int32),
            pltpu.VMEM((NPIECE, 128), jnp.int32),
            pltpu.VMEM((TPAD,), jnp.float32),
            pltpu.VMEM((TPAD,), jnp.float32),
            pltpu.VMEM((3, 128, H), jnp.float32),
            pltpu.SemaphoreType.DMA,
            pltpu.SemaphoreType.DMA,
        ],
        compiler_params=_sc_params,
    )(node_ids, timestamps, last_update, new_h, mem_ref)


def kernel(mem, node_ids, messages, timestamps, last_update,
           W_ih, W_hh, b_ih, b_hh):
    mem_ref = jax.new_ref(mem)
    ids2d = node_ids.reshape(128, 128)
    h = _sc_gather(mem, ids2d)
    new_h = _tc_gru(h, messages, W_ih, W_hh, b_ih, b_hh)
    lu_new = _sc_scatter_inplace(node_ids, timestamps, last_update, new_h,
                                 mem_ref)
    return jax.freeze(mem_ref), lu_new
